# Initial kernel scaffold; baseline (speedup 1.0000x reference)
import jax
import jax.numpy as jnp
from jax.experimental import pallas as pl


def kernel(input_ids, entity_ids, token_table, lkg_table, W, b):
    raise NotImplementedError("write your pallas kernel here")


# speedup vs baseline: 30.3072x; 30.3072x over previous
import functools

import jax
import jax.numpy as jnp
from jax import lax
from jax.experimental import pallas as pl
from jax.experimental.pallas import tpu as pltpu
from jax.experimental.pallas import tpu_sc as plsc

D = 32
VOCAB_N = 1_000_000
BATCH = 4096
SEQ = 200
N = BATCH * SEQ
NW = 32
PER_W = N // NW
CHUNK = 128
NCHUNK = PER_W // CHUNK
K_WAVE = 8
NWAVE = NCHUNK // K_WAVE

BLKC = 8192


def _proj_body(t1_ref, t2_ref, w1_ref, w2_ref, b_ref, p1_ref, p2_ref):
    dn = (((0,), (0,)), ((), ()))
    p1 = lax.dot_general(t1_ref[...], w1_ref[...], dn,
                         preferred_element_type=jnp.float32)
    p2 = lax.dot_general(t2_ref[...], w2_ref[...], dn,
                         preferred_element_type=jnp.float32)
    p2 = p2 + b_ref[...]
    r1 = p1.reshape(BLKC // 4, 4, D)
    r2 = p2.reshape(BLKC // 4, 4, D)
    p1_ref[...] = jnp.concatenate([r1[:, u, :] for u in range(4)], axis=1)
    p2_ref[...] = jnp.concatenate([r2[:, u, :] for u in range(4)], axis=1)


_proj = pl.pallas_call(
    _proj_body,
    grid=(pl.cdiv(VOCAB_N, BLKC),),
    in_specs=[
        pl.BlockSpec((D, BLKC), lambda i: (0, i)),
        pl.BlockSpec((D, BLKC), lambda i: (0, i)),
        pl.BlockSpec((D, D), lambda i: (0, 0)),
        pl.BlockSpec((D, D), lambda i: (0, 0)),
        pl.BlockSpec((1, D), lambda i: (0, 0)),
    ],
    out_specs=[
        pl.BlockSpec((BLKC // 4, 128), lambda i: (i, 0)),
        pl.BlockSpec((BLKC // 4, 128), lambda i: (i, 0)),
    ],
    out_shape=[
        jax.ShapeDtypeStruct((VOCAB_N // 4, 128), jnp.float32),
        jax.ShapeDtypeStruct((VOCAB_N // 4, 128), jnp.float32),
    ],
)

_mesh = plsc.VectorSubcoreMesh(core_axis_name="c", subcore_axis_name="s")


@functools.partial(
    pl.kernel,
    out_type=jax.ShapeDtypeStruct((N, D), jnp.float32),
    mesh=_mesh,
    scratch_types=[
        pltpu.VMEM((NCHUNK, CHUNK), jnp.int32),
        pltpu.VMEM((NCHUNK, CHUNK), jnp.int32),
        pltpu.VMEM((2, K_WAVE, CHUNK, D), jnp.float32),
        pltpu.SemaphoreType.DMA,
        pltpu.SemaphoreType.DMA,
    ],
    compiler_params=pltpu.CompilerParams(use_tc_tiling_on_sc=False),
)
def _sc_gather_add(p1, p2, ids, eids, out, idx_v, eidx_v, bufs, gsem, wsem):
    wid = lax.axis_index("s") * 2 + lax.axis_index("c")
    pltpu.sync_copy(ids.at[wid], idx_v)
    pltpu.sync_copy(eids.at[wid], eidx_v)
    base = wid * PER_W

    def wave(w, _):
        j0 = w * K_WAVE
        s = lax.rem(w, 2)
        bset = bufs.at[s]

        @pl.when(w >= 2)
        def _():
            for u in range(K_WAVE):
                pltpu.make_async_copy(
                    bset.at[u], out.at[pl.ds(base, CHUNK)], wsem
                ).wait()

        for u in range(K_WAVE):
            pltpu.async_copy(p1.at[idx_v.at[j0 + u]], bset.at[u], gsem)
        for u in range(K_WAVE):
            pltpu.make_async_copy(
                p1.at[idx_v.at[j0 + u]], bset.at[u], gsem
            ).wait()
        for u in range(K_WAVE):
            pltpu.async_copy(p2.at[eidx_v.at[j0 + u]], bset.at[u], gsem,
                             add=True)
        for u in range(K_WAVE):
            pltpu.make_async_copy(
                p2.at[eidx_v.at[j0 + u]], bset.at[u], gsem
            ).wait()
        for u in range(K_WAVE):
            pltpu.async_copy(
                bset.at[u],
                out.at[pl.ds(base + (j0 + u) * CHUNK, CHUNK)],
                wsem,
            )
        return 0

    lax.fori_loop(0, NWAVE, wave, 0)

    for u in range(2 * K_WAVE):
        pltpu.make_async_copy(
            bufs.at[0].at[0], out.at[pl.ds(base, CHUNK)], wsem
        ).wait()


def kernel(input_ids, entity_ids, token_table, lkg_table, W, b):
    ids = input_ids.reshape(NW, NCHUNK, CHUNK).astype(jnp.int32)
    eids = entity_ids.reshape(NW, NCHUNK, CHUNK).astype(jnp.int32)
    w1t = W[:, :D].T
    w2t = W[:, D:].T
    p1p, p2p = _proj(token_table.T, lkg_table.T, w1t, w2t, b.reshape(1, D))
    p1 = p1p.reshape(VOCAB_N, D)
    p2 = p2p.reshape(VOCAB_N, D)
    out = _sc_gather_add(p1, p2, ids, eids)
    return out.reshape(BATCH, SEQ, D)


# speedup vs baseline: 52.6526x; 1.7373x over previous
import functools

import jax
import jax.numpy as jnp
from jax import lax
from jax.experimental import pallas as pl
from jax.experimental.pallas import tpu as pltpu
from jax.experimental.pallas import tpu_sc as plsc

D = 32
VOCAB_N = 1_000_000
BATCH = 4096
SEQ = 200
N = BATCH * SEQ
NW = 32
PER_W = N // NW
CHUNK = 128
NCHUNK = PER_W // CHUNK
K_WAVE = 8
NWAVE = NCHUNK // K_WAVE

BLKC = 8192
NBLK = pl.cdiv(VOCAB_N, BLKC)
QBLK = BLKC // 4
VPAD = NBLK * BLKC


def _proj_body(t1_ref, t2_ref, y1_ref, y2_ref, b_ref, p1_ref, p2_ref):
    dn = (((0,), (0,)), ((), ()))
    x1 = jnp.concatenate(
        [t1_ref[:, u * QBLK:(u + 1) * QBLK] for u in range(4)], axis=0)
    x2 = jnp.concatenate(
        [t2_ref[:, u * QBLK:(u + 1) * QBLK] for u in range(4)], axis=0)
    p1 = lax.dot_general(x1, y1_ref[...], dn,
                         preferred_element_type=jnp.float32)
    p2 = lax.dot_general(x2, y2_ref[...], dn,
                         preferred_element_type=jnp.float32)
    p1_ref[...] = p1
    p2_ref[...] = p2 + b_ref[...]


_proj = pl.pallas_call(
    _proj_body,
    grid=(NBLK,),
    in_specs=[
        pl.BlockSpec((D, BLKC), lambda i: (0, i)),
        pl.BlockSpec((D, BLKC), lambda i: (0, i)),
        pl.BlockSpec((128, 128), lambda i: (0, 0)),
        pl.BlockSpec((128, 128), lambda i: (0, 0)),
        pl.BlockSpec((1, 128), lambda i: (0, 0)),
    ],
    out_specs=[
        pl.BlockSpec((QBLK, 128), lambda i: (i, 0)),
        pl.BlockSpec((QBLK, 128), lambda i: (i, 0)),
    ],
    out_shape=[
        jax.ShapeDtypeStruct((VPAD // 4, 128), jnp.float32),
        jax.ShapeDtypeStruct((VPAD // 4, 128), jnp.float32),
    ],
)

_mesh = plsc.VectorSubcoreMesh(core_axis_name="c", subcore_axis_name="s")


@functools.partial(
    pl.kernel,
    out_type=jax.ShapeDtypeStruct((N, D), jnp.float32),
    mesh=_mesh,
    scratch_types=[
        pltpu.VMEM((NCHUNK, CHUNK), jnp.int32),
        pltpu.VMEM((NCHUNK, CHUNK), jnp.int32),
        pltpu.VMEM((2, K_WAVE, CHUNK, D), jnp.float32),
        pltpu.SemaphoreType.DMA,
        pltpu.SemaphoreType.DMA,
    ],
    compiler_params=pltpu.CompilerParams(use_tc_tiling_on_sc=False),
)
def _sc_gather_add(p1, p2, ids, eids, out, idx_v, eidx_v, bufs, gsem, wsem):
    wid = lax.axis_index("s") * 2 + lax.axis_index("c")
    pltpu.sync_copy(ids.at[wid], idx_v)
    pltpu.sync_copy(eids.at[wid], eidx_v)
    base = wid * PER_W

    def wave(w, _):
        j0 = w * K_WAVE
        s = lax.rem(w, 2)
        bset = bufs.at[s]

        @pl.when(w >= 2)
        def _():
            for u in range(K_WAVE):
                pltpu.make_async_copy(
                    bset.at[u], out.at[pl.ds(base, CHUNK)], wsem
                ).wait()

        for u in range(K_WAVE):
            pltpu.async_copy(p1.at[idx_v.at[j0 + u]], bset.at[u], gsem)
        for u in range(K_WAVE):
            pltpu.make_async_copy(
                p1.at[idx_v.at[j0 + u]], bset.at[u], gsem
            ).wait()
        for u in range(K_WAVE):
            pltpu.async_copy(p2.at[eidx_v.at[j0 + u]], bset.at[u], gsem,
                             add=True)
        for u in range(K_WAVE):
            pltpu.make_async_copy(
                p2.at[eidx_v.at[j0 + u]], bset.at[u], gsem
            ).wait()
        for u in range(K_WAVE):
            pltpu.async_copy(
                bset.at[u],
                out.at[pl.ds(base + (j0 + u) * CHUNK, CHUNK)],
                wsem,
            )
        return 0

    lax.fori_loop(0, NWAVE, wave, 0)

    for u in range(2 * K_WAVE):
        pltpu.make_async_copy(
            bufs.at[0].at[0], out.at[pl.ds(base, CHUNK)], wsem
        ).wait()


def _pack_ids(raw):
    r = raw.reshape(-1).astype(jnp.int32)
    m = ((r >> 13) << 13) + ((r & (QBLK - 1)) << 2) + ((r >> 11) & 3)
    return m.reshape(NW, NCHUNK, CHUNK)


def kernel(input_ids, entity_ids, token_table, lkg_table, W, b):
    ids = _pack_ids(input_ids)
    eids = _pack_ids(entity_ids)
    eye4 = jnp.eye(4, dtype=jnp.float32)
    y1 = jnp.kron(eye4, W[:, :D].T)
    y2 = jnp.kron(eye4, W[:, D:].T)
    b128 = jnp.tile(b, 4).reshape(1, 128)
    p1p, p2p = _proj(token_table.T, lkg_table.T, y1, y2, b128)
    p1 = p1p.reshape(VPAD, D)
    p2 = p2p.reshape(VPAD, D)
    out = _sc_gather_add(p1, p2, ids, eids)
    return out.reshape(BATCH, SEQ, D)
